# baseline (speedup 1.0000x reference)
import functools

import jax
import jax.numpy as jnp
from jax import lax
from jax.experimental import pallas as pl
from jax.experimental.pallas import tpu as pltpu
from jax.experimental.pallas import tpu_sc as plsc

_N_NODES = 10000
_N_EDGES = 320000
_D_NODE = 128
_D_EDGE = 16
_D_GLOBAL = 32


_EB = 32000


def _edge_mlp_body(x_ref, w_ref, b_ref, o_ref, cs_ref):
    i = pl.program_id(0)
    y = jnp.dot(w_ref[...], x_ref[...], preferred_element_type=jnp.float32)
    y = jnp.maximum(y + b_ref[...], 0.0)
    o_ref[...] = y

    @pl.when(i == 0)
    def _():
        cs_ref[...] = jnp.zeros_like(cs_ref)

    cs_ref[...] += jnp.sum(y, axis=1, keepdims=True)


def _edge_mlp(edges_t, w_t, b_col):
    grid = _N_EDGES // _EB
    return pl.pallas_call(
        _edge_mlp_body,
        grid=(grid,),
        in_specs=[
            pl.BlockSpec((16, _EB), lambda i: (0, i)),
            pl.BlockSpec((16, 16), lambda i: (0, 0)),
            pl.BlockSpec((16, 1), lambda i: (0, 0)),
        ],
        out_specs=[
            pl.BlockSpec((16, _EB), lambda i: (0, i)),
            pl.BlockSpec((16, 1), lambda i: (0, 0)),
        ],
        out_shape=[
            jax.ShapeDtypeStruct((16, _N_EDGES), jnp.float32),
            jax.ShapeDtypeStruct((16, 1), jnp.float32),
        ],
    )(edges_t, w_t, b_col)


_NC = 2
_NS = 16
_NW = _NC * _NS
_N_CHUNKS = _N_EDGES // 128
_G = 8
_HI = 24
_STRIPE = 624


def _seg_sum_body(yt_hbm, dst_hbm, zero_hbm, out_hbm,
                  idx_v, slab_v, trans_v, accum_sh, sem):
    cid = lax.axis_index("c")
    sid = lax.axis_index("s")
    wid = cid * _NS + sid
    iota = lax.iota(jnp.int32, 16)

    @pl.when(sid < _NS - 1)
    def _():
        pltpu.sync_copy(zero_hbm.at[pl.ds(sid * _STRIPE, _STRIPE)],
                        accum_sh.at[pl.ds(sid * _STRIPE, _STRIPE)])

    @pl.when(sid == _NS - 1)
    def _():
        pltpu.sync_copy(zero_hbm.at[pl.ds((_NS - 1) * _STRIPE, 640)],
                        accum_sh.at[pl.ds((_NS - 1) * _STRIPE, 640)])

    plsc.subcore_barrier()

    def transpose_block(n_chunks):
        for f0 in range(16):
            rowi = jnp.remainder(f0 + iota, 16)

            def body(pb2, _):
                for u in range(8):
                    coli = (pb2 * 8 + u) * 16 + iota
                    v = plsc.load_gather(slab_v, [rowi, coli])
                    plsc.store_scatter(trans_v, [coli, rowi], v)
                return 0

            lax.fori_loop(0, n_chunks, body, 0, unroll=False)

    def window(base_chunk):
        e0 = base_chunk * 128
        copies = [pltpu.make_async_copy(
            yt_hbm.at[f, pl.ds(e0, _G * 128)], slab_v.at[f], sem)
            for f in range(16)]
        for c in copies:
            c.start()
        for c in copies:
            c.wait()
        pltpu.sync_copy(dst_hbm.at[pl.ds(base_chunk, _G)], idx_v)
        transpose_block(_G)
        for j in range(_G):
            pltpu.sync_copy(trans_v.at[pl.ds(j * 128, 128)],
                            accum_sh.at[idx_v.at[j]], add=True)

    base = jnp.where(wid < _HI, wid * 10 * _G,
                     (_HI * 10 + (wid - _HI) * 9) * _G)
    n_groups = jnp.where(wid < _HI, 10, 9)

    def group_body(g, _):
        window(base + g * _G)
        return 0

    lax.fori_loop(0, n_groups, group_body, 0, unroll=False)

    @pl.when(wid < _N_CHUNKS - 312 * _G)
    def _():
        c = 312 * _G + wid
        for f in range(16):
            pltpu.sync_copy(yt_hbm.at[f, pl.ds(c * 128, 128)],
                            slab_v.at[f, pl.ds(0, 128)])
        pltpu.sync_copy(dst_hbm.at[pl.ds(312 * _G, _G)], idx_v)
        transpose_block(1)
        pltpu.sync_copy(trans_v.at[pl.ds(0, 128)],
                        accum_sh.at[idx_v.at[wid]], add=True)

    plsc.subcore_barrier()

    @pl.when(sid < _NS - 1)
    def _():
        pltpu.sync_copy(accum_sh.at[pl.ds(sid * _STRIPE, _STRIPE)],
                        out_hbm.at[cid].at[pl.ds(sid * _STRIPE, _STRIPE)])

    @pl.when(sid == _NS - 1)
    def _():
        pltpu.sync_copy(accum_sh.at[pl.ds((_NS - 1) * _STRIPE, 640)],
                        out_hbm.at[cid].at[pl.ds((_NS - 1) * _STRIPE, 640)])


def _segment_sum_sc(yt, dst2d, zeros_hbm):
    mesh = plsc.VectorSubcoreMesh(core_axis_name="c", subcore_axis_name="s",
                                  num_cores=_NC, num_subcores=_NS)
    f = pl.kernel(
        _seg_sum_body,
        out_type=jax.ShapeDtypeStruct((_NC, _N_NODES, _D_EDGE), jnp.float32),
        mesh=mesh,
        scratch_types=[
            pltpu.VMEM((_G, 128), jnp.int32),
            pltpu.VMEM((16, _G * 128), jnp.float32),
            pltpu.VMEM((_G * 128, _D_EDGE), jnp.float32),
            pltpu.VMEM_SHARED((_N_NODES, _D_EDGE), jnp.float32),
            pltpu.SemaphoreType.DMA,
        ],
        name="segment_sum_sc",
        compiler_params=pltpu.CompilerParams(use_tc_tiling_on_sc=False,
                                             needs_layout_passes=False),
    )
    return f(yt, dst2d, zeros_hbm)


_NB = 1000


def _node_mlp_body(nf_ref, p_ref, wn1_ref, wn2_ref, bn_ref,
                   ecs_ref, gf_ref, wg1_ref, wg2_ref, wg3_ref, bg_ref,
                   out_ref, g_ref, ns_ref):
    i = pl.program_id(0)
    ne = p_ref[0] + p_ref[1]
    h = jnp.dot(nf_ref[...], wn1_ref[...], preferred_element_type=jnp.float32)
    h += jnp.dot(ne, wn2_ref[...], preferred_element_type=jnp.float32)
    h = jnp.maximum(h + bn_ref[...], 0.0)
    out_ref[...] = h

    @pl.when(i == 0)
    def _():
        ns_ref[...] = jnp.zeros_like(ns_ref)

    ns_ref[...] += jnp.sum(h, axis=0, keepdims=True)

    @pl.when(i == pl.num_programs(0) - 1)
    def _():
        gn = ns_ref[...] * (1.0 / _N_NODES)
        ge = ecs_ref[...] * (1.0 / _N_EDGES)
        g = jnp.dot(gf_ref[...], wg1_ref[...],
                    preferred_element_type=jnp.float32)
        g += jnp.dot(gn, wg2_ref[...], preferred_element_type=jnp.float32)
        g += lax.dot_general(ge, wg3_ref[...], (((0,), (0,)), ((), ())),
                             preferred_element_type=jnp.float32)
        g_ref[...] = jnp.maximum(g + bg_ref[...], 0.0)


def _node_mlp(node_features, partials, wn1, wn2, bn, ecs, gf,
              wg1, wg2, wg3, bg):
    grid = _N_NODES // _NB
    return pl.pallas_call(
        _node_mlp_body,
        grid=(grid,),
        in_specs=[
            pl.BlockSpec((_NB, 128), lambda i: (i, 0)),
            pl.BlockSpec((_NC, _NB, 16), lambda i: (0, i, 0)),
            pl.BlockSpec((128, 128), lambda i: (0, 0)),
            pl.BlockSpec((16, 128), lambda i: (0, 0)),
            pl.BlockSpec((1, 128), lambda i: (0, 0)),
            pl.BlockSpec((16, 1), lambda i: (0, 0)),
            pl.BlockSpec((1, 32), lambda i: (0, 0)),
            pl.BlockSpec((32, 32), lambda i: (0, 0)),
            pl.BlockSpec((128, 32), lambda i: (0, 0)),
            pl.BlockSpec((16, 32), lambda i: (0, 0)),
            pl.BlockSpec((1, 32), lambda i: (0, 0)),
        ],
        out_specs=[
            pl.BlockSpec((_NB, 128), lambda i: (i, 0)),
            pl.BlockSpec((1, 32), lambda i: (0, 0)),
            pl.BlockSpec((1, 128), lambda i: (0, 0)),
        ],
        out_shape=[
            jax.ShapeDtypeStruct((_N_NODES, 128), jnp.float32),
            jax.ShapeDtypeStruct((1, 32), jnp.float32),
            jax.ShapeDtypeStruct((1, 128), jnp.float32),
        ],
    )(node_features, partials, wn1, wn2, bn, ecs, gf, wg1, wg2, wg3, bg)


def kernel(node_features, edges_features, global_features, edge_dst,
           W_e, b_e, W_n, b_n, W_g, b_g):
    edges_t = edges_features.T
    yt, ecs = _edge_mlp(edges_t, W_e.T, b_e.reshape(_D_EDGE, 1))
    new_edge = yt.T

    dst2d = jnp.concatenate(
        [edge_dst, jnp.zeros((512,), jnp.int32)]).reshape(_N_CHUNKS + 4, 128)
    zeros_hbm = jnp.zeros((_N_NODES, _D_EDGE), jnp.float32)
    partials = _segment_sum_sc(yt, dst2d, zeros_hbm)

    new_node, g, _ = _node_mlp(
        node_features, partials,
        W_n[:_D_NODE], W_n[_D_NODE:], b_n.reshape(1, 128),
        ecs, global_features.reshape(1, _D_GLOBAL),
        W_g[:_D_GLOBAL], W_g[_D_GLOBAL:_D_GLOBAL + _D_NODE],
        W_g[_D_GLOBAL + _D_NODE:], b_g.reshape(1, _D_GLOBAL))

    return (new_node, new_edge, g.reshape(_D_GLOBAL))


# speedup vs baseline: 1.1487x; 1.1487x over previous
import functools

import jax
import jax.numpy as jnp
from jax import lax
from jax.experimental import pallas as pl
from jax.experimental.pallas import tpu as pltpu
from jax.experimental.pallas import tpu_sc as plsc

_N_NODES = 10000
_N_EDGES = 320000
_D_NODE = 128
_D_EDGE = 16
_D_GLOBAL = 32


_EB = 32000


def _edge_mlp_body(x_ref, w_ref, b_ref, o_ref, cs_ref):
    i = pl.program_id(0)
    y = jnp.dot(w_ref[...], x_ref[...], preferred_element_type=jnp.float32)
    y = jnp.maximum(y + b_ref[...], 0.0)
    o_ref[...] = y

    @pl.when(i == 0)
    def _():
        cs_ref[...] = jnp.zeros_like(cs_ref)

    cs_ref[...] += jnp.sum(y, axis=1, keepdims=True)


def _edge_mlp(edges_t, w_t, b_col):
    grid = _N_EDGES // _EB
    return pl.pallas_call(
        _edge_mlp_body,
        grid=(grid,),
        in_specs=[
            pl.BlockSpec((16, _EB), lambda i: (0, i)),
            pl.BlockSpec((16, 16), lambda i: (0, 0)),
            pl.BlockSpec((16, 1), lambda i: (0, 0)),
        ],
        out_specs=[
            pl.BlockSpec((16, _EB), lambda i: (0, i)),
            pl.BlockSpec((16, 1), lambda i: (0, 0)),
        ],
        out_shape=[
            jax.ShapeDtypeStruct((16, _N_EDGES), jnp.float32),
            jax.ShapeDtypeStruct((16, 1), jnp.float32),
        ],
    )(edges_t, w_t, b_col)


_NC = 2
_NS = 16
_NW = _NC * _NS
_N_CHUNKS = _N_EDGES // 128
_G = 8
_HI = 24
_STRIPE = 624


def _seg_sum_body(yt_hbm, dst_hbm, zero_hbm, out_hbm,
                  idx_v, slab_a, slab_b, trans_v, accum_sh, sem_a, sem_b):
    cid = lax.axis_index("c")
    sid = lax.axis_index("s")
    wid = cid * _NS + sid
    iota = lax.iota(jnp.int32, 16)

    @pl.when(sid < _NS - 1)
    def _():
        pltpu.sync_copy(zero_hbm.at[pl.ds(sid * _STRIPE, _STRIPE)],
                        accum_sh.at[pl.ds(sid * _STRIPE, _STRIPE)])

    @pl.when(sid == _NS - 1)
    def _():
        pltpu.sync_copy(zero_hbm.at[pl.ds((_NS - 1) * _STRIPE, 640)],
                        accum_sh.at[pl.ds((_NS - 1) * _STRIPE, 640)])

    plsc.subcore_barrier()

    def transpose_block(slab, n_chunks):
        for f0 in range(16):
            rowi = jnp.remainder(f0 + iota, 16)

            def body(pb2, _):
                for u in range(4):
                    coli = (pb2 * 4 + u) * 16 + iota
                    v = plsc.load_gather(slab, [rowi, coli])
                    plsc.store_scatter(trans_v, [coli, rowi], v)
                return 0

            lax.fori_loop(0, n_chunks * 2, body, 0, unroll=False)

    def stage_copies(slab, semx, base_chunk):
        e0 = base_chunk * 128
        return [pltpu.make_async_copy(
            yt_hbm.at[f, pl.ds(e0, _G * 128)], slab.at[f], semx)
            for f in range(16)]

    base = jnp.where(wid < _HI, wid * 10 * _G,
                     (_HI * 10 + (wid - _HI) * 9) * _G)
    n_groups = jnp.where(wid < _HI, 10, 9)

    def process(slab, semx, slab_n, sem_n, g):
        base_chunk = base + g * _G
        for c in stage_copies(slab, semx, base_chunk):
            c.wait()

        @pl.when(g + 1 < n_groups)
        def _():
            for c in stage_copies(slab_n, sem_n, base_chunk + _G):
                c.start()

        pltpu.sync_copy(dst_hbm.at[pl.ds(base_chunk, _G)], idx_v)
        transpose_block(slab, _G)
        for j in range(_G):
            pltpu.sync_copy(trans_v.at[pl.ds(j * 128, 128)],
                            accum_sh.at[idx_v.at[j]], add=True)

    for c in stage_copies(slab_a, sem_a, base):
        c.start()

    def group_body(g, _):
        @pl.when((g & 1) == 0)
        def _():
            process(slab_a, sem_a, slab_b, sem_b, g)

        @pl.when((g & 1) == 1)
        def _():
            process(slab_b, sem_b, slab_a, sem_a, g)

        return 0

    lax.fori_loop(0, n_groups, group_body, 0, unroll=False)

    @pl.when(wid < _N_CHUNKS - 312 * _G)
    def _():
        c = 312 * _G + wid
        for f in range(16):
            pltpu.sync_copy(yt_hbm.at[f, pl.ds(c * 128, 128)],
                            slab_a.at[f, pl.ds(0, 128)])
        pltpu.sync_copy(dst_hbm.at[pl.ds(312 * _G, _G)], idx_v)
        transpose_block(slab_a, 1)
        pltpu.sync_copy(trans_v.at[pl.ds(0, 128)],
                        accum_sh.at[idx_v.at[wid]], add=True)

    plsc.subcore_barrier()

    @pl.when(sid < _NS - 1)
    def _():
        pltpu.sync_copy(accum_sh.at[pl.ds(sid * _STRIPE, _STRIPE)],
                        out_hbm.at[cid].at[pl.ds(sid * _STRIPE, _STRIPE)])

    @pl.when(sid == _NS - 1)
    def _():
        pltpu.sync_copy(accum_sh.at[pl.ds((_NS - 1) * _STRIPE, 640)],
                        out_hbm.at[cid].at[pl.ds((_NS - 1) * _STRIPE, 640)])


def _segment_sum_sc(yt, dst2d, zeros_hbm):
    mesh = plsc.VectorSubcoreMesh(core_axis_name="c", subcore_axis_name="s",
                                  num_cores=_NC, num_subcores=_NS)
    f = pl.kernel(
        _seg_sum_body,
        out_type=jax.ShapeDtypeStruct((_NC, _N_NODES, _D_EDGE), jnp.float32),
        mesh=mesh,
        scratch_types=[
            pltpu.VMEM((_G, 128), jnp.int32),
            pltpu.VMEM((16, _G * 128), jnp.float32),
            pltpu.VMEM((16, _G * 128), jnp.float32),
            pltpu.VMEM((_G * 128, _D_EDGE), jnp.float32),
            pltpu.VMEM_SHARED((_N_NODES, _D_EDGE), jnp.float32),
            pltpu.SemaphoreType.DMA,
            pltpu.SemaphoreType.DMA,
        ],
        name="segment_sum_sc",
        compiler_params=pltpu.CompilerParams(use_tc_tiling_on_sc=False,
                                             needs_layout_passes=False),
    )
    return f(yt, dst2d, zeros_hbm)


_NB = 1000


def _node_mlp_body(nf_ref, p_ref, wn1_ref, wn2_ref, bn_ref,
                   ecs_ref, gf_ref, wg1_ref, wg2_ref, wg3_ref, bg_ref,
                   out_ref, g_ref, ns_ref):
    i = pl.program_id(0)
    ne = p_ref[0] + p_ref[1]
    h = jnp.dot(nf_ref[...], wn1_ref[...], preferred_element_type=jnp.float32)
    h += jnp.dot(ne, wn2_ref[...], preferred_element_type=jnp.float32)
    h = jnp.maximum(h + bn_ref[...], 0.0)
    out_ref[...] = h

    @pl.when(i == 0)
    def _():
        ns_ref[...] = jnp.zeros_like(ns_ref)

    ns_ref[...] += jnp.sum(h, axis=0, keepdims=True)

    @pl.when(i == pl.num_programs(0) - 1)
    def _():
        gn = ns_ref[...] * (1.0 / _N_NODES)
        ge = ecs_ref[...] * (1.0 / _N_EDGES)
        g = jnp.dot(gf_ref[...], wg1_ref[...],
                    preferred_element_type=jnp.float32)
        g += jnp.dot(gn, wg2_ref[...], preferred_element_type=jnp.float32)
        g += lax.dot_general(ge, wg3_ref[...], (((0,), (0,)), ((), ())),
                             preferred_element_type=jnp.float32)
        g_ref[...] = jnp.maximum(g + bg_ref[...], 0.0)


def _node_mlp(node_features, partials, wn1, wn2, bn, ecs, gf,
              wg1, wg2, wg3, bg):
    grid = _N_NODES // _NB
    return pl.pallas_call(
        _node_mlp_body,
        grid=(grid,),
        in_specs=[
            pl.BlockSpec((_NB, 128), lambda i: (i, 0)),
            pl.BlockSpec((_NC, _NB, 16), lambda i: (0, i, 0)),
            pl.BlockSpec((128, 128), lambda i: (0, 0)),
            pl.BlockSpec((16, 128), lambda i: (0, 0)),
            pl.BlockSpec((1, 128), lambda i: (0, 0)),
            pl.BlockSpec((16, 1), lambda i: (0, 0)),
            pl.BlockSpec((1, 32), lambda i: (0, 0)),
            pl.BlockSpec((32, 32), lambda i: (0, 0)),
            pl.BlockSpec((128, 32), lambda i: (0, 0)),
            pl.BlockSpec((16, 32), lambda i: (0, 0)),
            pl.BlockSpec((1, 32), lambda i: (0, 0)),
        ],
        out_specs=[
            pl.BlockSpec((_NB, 128), lambda i: (i, 0)),
            pl.BlockSpec((1, 32), lambda i: (0, 0)),
            pl.BlockSpec((1, 128), lambda i: (0, 0)),
        ],
        out_shape=[
            jax.ShapeDtypeStruct((_N_NODES, 128), jnp.float32),
            jax.ShapeDtypeStruct((1, 32), jnp.float32),
            jax.ShapeDtypeStruct((1, 128), jnp.float32),
        ],
    )(node_features, partials, wn1, wn2, bn, ecs, gf, wg1, wg2, wg3, bg)


def kernel(node_features, edges_features, global_features, edge_dst,
           W_e, b_e, W_n, b_n, W_g, b_g):
    edges_t = edges_features.T
    yt, ecs = _edge_mlp(edges_t, W_e.T, b_e.reshape(_D_EDGE, 1))
    new_edge = yt.T

    dst2d = jnp.concatenate(
        [edge_dst, jnp.zeros((512,), jnp.int32)]).reshape(_N_CHUNKS + 4, 128)
    zeros_hbm = jnp.zeros((_N_NODES, _D_EDGE), jnp.float32)
    partials = _segment_sum_sc(yt, dst2d, zeros_hbm)

    new_node, g, _ = _node_mlp(
        node_features, partials,
        W_n[:_D_NODE], W_n[_D_NODE:], b_n.reshape(1, 128),
        ecs, global_features.reshape(1, _D_GLOBAL),
        W_g[:_D_GLOBAL], W_g[_D_GLOBAL:_D_GLOBAL + _D_NODE],
        W_g[_D_GLOBAL + _D_NODE:], b_g.reshape(1, _D_GLOBAL))

    return (new_node, new_edge, g.reshape(_D_GLOBAL))


# speedup vs baseline: 1.4259x; 1.2413x over previous
import jax
import jax.numpy as jnp
from jax import lax
from jax.experimental import pallas as pl
from jax.experimental.pallas import tpu as pltpu
from jax.experimental.pallas import tpu_sc as plsc

_N_NODES = 10000
_N_EDGES = 320000
_D_NODE = 128
_D_EDGE = 16
_D_GLOBAL = 32


_EB = 32000


def _edge_mlp_body(x_ref, w_ref, b_ref, o_ref, cs_ref):
    i = pl.program_id(0)
    y = jnp.dot(w_ref[...], x_ref[...], preferred_element_type=jnp.float32)
    y = jnp.maximum(y + b_ref[...], 0.0)
    o_ref[...] = y

    @pl.when(i == 0)
    def _():
        cs_ref[...] = jnp.zeros_like(cs_ref)

    cs_ref[...] += jnp.sum(y, axis=1, keepdims=True)


def _edge_mlp(edges_t, w_t, b_col):
    grid = _N_EDGES // _EB
    return pl.pallas_call(
        _edge_mlp_body,
        grid=(grid,),
        in_specs=[
            pl.BlockSpec((16, _EB), lambda i: (0, i)),
            pl.BlockSpec((16, 16), lambda i: (0, 0)),
            pl.BlockSpec((16, 1), lambda i: (0, 0)),
        ],
        out_specs=[
            pl.BlockSpec((16, _EB), lambda i: (0, i)),
            pl.BlockSpec((16, 1), lambda i: (0, 0)),
        ],
        out_shape=[
            jax.ShapeDtypeStruct((16, _N_EDGES), jnp.float32),
            jax.ShapeDtypeStruct((16, 1), jnp.float32),
        ],
    )(edges_t, w_t, b_col)


_NC = 2
_NS = 16
_NW = _NC * _NS
_N_CHUNKS = _N_EDGES // 128
_G = 8
_HI = 24
_STRIPE = 624


def _seg_sum_body(yt_hbm, dst_hbm, zero_hbm, out_hbm,
                  idx_v, slab_a, slab_b, trans_v, accum_sh, sem_a, sem_b):
    cid = lax.axis_index("c")
    sid = lax.axis_index("s")
    wid = cid * _NS + sid
    iota = lax.iota(jnp.int32, 16)

    @pl.when(sid < _NS - 1)
    def _():
        pltpu.sync_copy(zero_hbm.at[pl.ds(sid * _STRIPE, _STRIPE)],
                        accum_sh.at[pl.ds(sid * _STRIPE, _STRIPE)])

    @pl.when(sid == _NS - 1)
    def _():
        pltpu.sync_copy(zero_hbm.at[pl.ds((_NS - 1) * _STRIPE, 640)],
                        accum_sh.at[pl.ds((_NS - 1) * _STRIPE, 640)])

    plsc.subcore_barrier()

    def transpose_block(slab, n_chunks):
        for f0 in range(16):
            rowi = jnp.remainder(f0 + iota, 16)

            @plsc.parallel_loop(0, n_chunks * 8, unroll=4)
            def _(pb):
                coli = pb * 16 + iota
                v = plsc.load_gather(slab, [rowi, coli])
                plsc.store_scatter(trans_v, [coli, rowi], v)

    def stage_copies(slab, semx, base_chunk):
        e0 = base_chunk * 128
        return [pltpu.make_async_copy(
            yt_hbm.at[f, pl.ds(e0, _G * 128)], slab.at[f], semx)
            for f in range(16)]

    base = jnp.where(wid < _HI, wid * 10 * _G,
                     (_HI * 10 + (wid - _HI) * 9) * _G)
    n_groups = jnp.where(wid < _HI, 10, 9)

    def process(slab, semx, slab_n, sem_n, g):
        base_chunk = base + g * _G
        for c in stage_copies(slab, semx, base_chunk):
            c.wait()

        @pl.when(g + 1 < n_groups)
        def _():
            for c in stage_copies(slab_n, sem_n, base_chunk + _G):
                c.start()

        pltpu.sync_copy(dst_hbm.at[pl.ds(base_chunk, _G)], idx_v)
        transpose_block(slab, _G)
        for j in range(_G):
            pltpu.sync_copy(trans_v.at[pl.ds(j * 128, 128)],
                            accum_sh.at[idx_v.at[j]], add=True)

    for c in stage_copies(slab_a, sem_a, base):
        c.start()

    def group_body(g, _):
        @pl.when((g & 1) == 0)
        def _():
            process(slab_a, sem_a, slab_b, sem_b, g)

        @pl.when((g & 1) == 1)
        def _():
            process(slab_b, sem_b, slab_a, sem_a, g)

        return 0

    lax.fori_loop(0, n_groups, group_body, 0, unroll=False)

    @pl.when(wid < _N_CHUNKS - 312 * _G)
    def _():
        c = 312 * _G + wid
        for f in range(16):
            pltpu.sync_copy(yt_hbm.at[f, pl.ds(c * 128, 128)],
                            slab_a.at[f, pl.ds(0, 128)])
        pltpu.sync_copy(dst_hbm.at[pl.ds(312 * _G, _G)], idx_v)
        transpose_block(slab_a, 1)
        pltpu.sync_copy(trans_v.at[pl.ds(0, 128)],
                        accum_sh.at[idx_v.at[wid]], add=True)

    plsc.subcore_barrier()

    @pl.when(sid < _NS - 1)
    def _():
        pltpu.sync_copy(accum_sh.at[pl.ds(sid * _STRIPE, _STRIPE)],
                        out_hbm.at[cid].at[pl.ds(sid * _STRIPE, _STRIPE)])

    @pl.when(sid == _NS - 1)
    def _():
        pltpu.sync_copy(accum_sh.at[pl.ds((_NS - 1) * _STRIPE, 640)],
                        out_hbm.at[cid].at[pl.ds((_NS - 1) * _STRIPE, 640)])


def _segment_sum_sc(yt, dst2d, zeros_hbm):
    mesh = plsc.VectorSubcoreMesh(core_axis_name="c", subcore_axis_name="s",
                                  num_cores=_NC, num_subcores=_NS)
    f = pl.kernel(
        _seg_sum_body,
        out_type=jax.ShapeDtypeStruct((_NC, _N_NODES, _D_EDGE), jnp.float32),
        mesh=mesh,
        scratch_types=[
            pltpu.VMEM((_G, 128), jnp.int32),
            pltpu.VMEM((16, _G * 128), jnp.float32),
            pltpu.VMEM((16, _G * 128), jnp.float32),
            pltpu.VMEM((_G * 128, _D_EDGE), jnp.float32),
            pltpu.VMEM_SHARED((_N_NODES, _D_EDGE), jnp.float32),
            pltpu.SemaphoreType.DMA,
            pltpu.SemaphoreType.DMA,
        ],
        name="segment_sum_sc",
        compiler_params=pltpu.CompilerParams(use_tc_tiling_on_sc=False,
                                             needs_layout_passes=False),
    )
    return f(yt, dst2d, zeros_hbm)


_NB = 1000


def _node_mlp_body(nf_ref, p_ref, wn1_ref, wn2_ref, bn_ref,
                   ecs_ref, gf_ref, wg1_ref, wg2_ref, wg3_ref, bg_ref,
                   out_ref, g_ref, ns_ref):
    i = pl.program_id(0)
    ne = p_ref[0] + p_ref[1]
    h = jnp.dot(nf_ref[...], wn1_ref[...], preferred_element_type=jnp.float32)
    h += jnp.dot(ne, wn2_ref[...], preferred_element_type=jnp.float32)
    h = jnp.maximum(h + bn_ref[...], 0.0)
    out_ref[...] = h

    @pl.when(i == 0)
    def _():
        ns_ref[...] = jnp.zeros_like(ns_ref)

    ns_ref[...] += jnp.sum(h, axis=0, keepdims=True)

    @pl.when(i == pl.num_programs(0) - 1)
    def _():
        gn = ns_ref[...] * (1.0 / _N_NODES)
        ge = ecs_ref[...] * (1.0 / _N_EDGES)
        g = jnp.dot(gf_ref[...], wg1_ref[...],
                    preferred_element_type=jnp.float32)
        g += jnp.dot(gn, wg2_ref[...], preferred_element_type=jnp.float32)
        g += lax.dot_general(ge, wg3_ref[...], (((0,), (0,)), ((), ())),
                             preferred_element_type=jnp.float32)
        g_ref[...] = jnp.maximum(g + bg_ref[...], 0.0)


def _node_mlp(node_features, partials, wn1, wn2, bn, ecs, gf,
              wg1, wg2, wg3, bg):
    grid = _N_NODES // _NB
    return pl.pallas_call(
        _node_mlp_body,
        grid=(grid,),
        in_specs=[
            pl.BlockSpec((_NB, 128), lambda i: (i, 0)),
            pl.BlockSpec((_NC, _NB, 16), lambda i: (0, i, 0)),
            pl.BlockSpec((128, 128), lambda i: (0, 0)),
            pl.BlockSpec((16, 128), lambda i: (0, 0)),
            pl.BlockSpec((1, 128), lambda i: (0, 0)),
            pl.BlockSpec((16, 1), lambda i: (0, 0)),
            pl.BlockSpec((1, 32), lambda i: (0, 0)),
            pl.BlockSpec((32, 32), lambda i: (0, 0)),
            pl.BlockSpec((128, 32), lambda i: (0, 0)),
            pl.BlockSpec((16, 32), lambda i: (0, 0)),
            pl.BlockSpec((1, 32), lambda i: (0, 0)),
        ],
        out_specs=[
            pl.BlockSpec((_NB, 128), lambda i: (i, 0)),
            pl.BlockSpec((1, 32), lambda i: (0, 0)),
            pl.BlockSpec((1, 128), lambda i: (0, 0)),
        ],
        out_shape=[
            jax.ShapeDtypeStruct((_N_NODES, 128), jnp.float32),
            jax.ShapeDtypeStruct((1, 32), jnp.float32),
            jax.ShapeDtypeStruct((1, 128), jnp.float32),
        ],
    )(node_features, partials, wn1, wn2, bn, ecs, gf, wg1, wg2, wg3, bg)


def kernel(node_features, edges_features, global_features, edge_dst,
           W_e, b_e, W_n, b_n, W_g, b_g):
    edges_t = edges_features.T
    yt, ecs = _edge_mlp(edges_t, W_e.T, b_e.reshape(_D_EDGE, 1))
    new_edge = yt.T

    dst2d = jnp.concatenate(
        [edge_dst, jnp.zeros((512,), jnp.int32)]).reshape(_N_CHUNKS + 4, 128)
    zeros_hbm = jnp.zeros((_N_NODES, _D_EDGE), jnp.float32)
    partials = _segment_sum_sc(yt, dst2d, zeros_hbm)

    new_node, g, _ = _node_mlp(
        node_features, partials,
        W_n[:_D_NODE], W_n[_D_NODE:], b_n.reshape(1, 128),
        ecs, global_features.reshape(1, _D_GLOBAL),
        W_g[:_D_GLOBAL], W_g[_D_GLOBAL:_D_GLOBAL + _D_NODE],
        W_g[_D_GLOBAL + _D_NODE:], b_g.reshape(1, _D_GLOBAL))

    return (new_node, new_edge, g.reshape(_D_GLOBAL))
